# Initial kernel scaffold; baseline (speedup 1.0000x reference)
import jax
import jax.numpy as jnp
from jax.experimental import pallas as pl


def kernel(center, context, neg_samples, in_embed_W, out_embed_W):
    raise NotImplementedError("write your pallas kernel here")


# speedup vs baseline: 4.5889x; 4.5889x over previous
import functools

import jax
import jax.numpy as jnp
from jax import lax
from jax.experimental import pallas as pl
from jax.experimental.pallas import tpu as pltpu
from jax.experimental.pallas import tpu_sc as plsc

VOCAB = 1000000
EMBED = 64
BATCH = 16384
NEG = 20
NPART = NEG + 1

_NC = 2
_NS = 16
_NW = _NC * _NS
_BPW = BATCH // _NW
_CHUNK = 128
_NCH = _BPW // _CHUNK


def _sc_gather_body(cidx, pidx, in_w, out_w, crows, prows, idxbuf, rowbuf,
                    isem, gsem, wsem):
    wid = lax.axis_index("s") * _NC + lax.axis_index("c")
    base4 = wid * _NCH
    baser = wid * _BPW

    tasks = [(in_w, cidx.at[pl.ds(base4, _NCH)], crows.at[pl.ds(baser, _BPW)])]
    for t in range(NPART):
        tasks.append((out_w, pidx.at[t, pl.ds(base4, _NCH)],
                      prows.at[t, pl.ds(baser, _BPW)]))
    n = len(tasks)

    idx_h = [None] * n
    wr_h = [None] * n
    idx_h[0] = pltpu.async_copy(tasks[0][1], idxbuf.at[0], isem.at[0])
    for t in range(n):
        s = t % 2
        idx_h[t].wait()
        if t + 1 < n:
            idx_h[t + 1] = pltpu.async_copy(tasks[t + 1][1], idxbuf.at[1 - s],
                                            isem.at[1 - s])
        if t >= 2:
            wr_h[t - 2].wait()
        gh = []
        for j in range(_NCH):
            gh.append(pltpu.async_copy(
                tasks[t][0].at[idxbuf.at[s, j]],
                rowbuf.at[s, pl.ds(j * _CHUNK, _CHUNK)],
                gsem.at[s]))
        for h in gh:
            h.wait()
        wr_h[t] = pltpu.async_copy(rowbuf.at[s], tasks[t][2], wsem.at[s])
    wr_h[n - 2].wait()
    wr_h[n - 1].wait()


@jax.jit
def _sc_gather(cidx, pidx, in_w, out_w):
    mesh = plsc.VectorSubcoreMesh(core_axis_name="c", subcore_axis_name="s")
    f = functools.partial(
        pl.kernel,
        mesh=mesh,
        compiler_params=pltpu.CompilerParams(use_tc_tiling_on_sc=False),
        out_type=(
            jax.ShapeDtypeStruct((BATCH, EMBED), jnp.float32),
            jax.ShapeDtypeStruct((NPART, BATCH, EMBED), jnp.float32),
        ),
        scratch_types=[
            pltpu.VMEM((2, _NCH, _CHUNK), jnp.int32),
            pltpu.VMEM((2, _BPW, EMBED), jnp.float32),
            pltpu.SemaphoreType.DMA((2,)),
            pltpu.SemaphoreType.DMA((2,)),
            pltpu.SemaphoreType.DMA((2,)),
        ],
    )(_sc_gather_body)
    return f(cidx, pidx, in_w, out_w)


_LOSS_C = 512
_LOSS_G = BATCH // _LOSS_C


def _loss_body(c_ref, p_ref, o_ref):
    i = pl.program_id(0)
    c = c_ref[...]
    p = p_ref[...]
    s = jnp.sum(p * c[None, :, :], axis=-1)
    sign = jnp.where(
        lax.broadcasted_iota(jnp.int32, s.shape, 0) == 0, 1.0, -1.0)
    term = jnp.log(jax.nn.sigmoid(sign * s) + 1e-10)
    tot = jnp.sum(term)

    @pl.when(i == 0)
    def _():
        o_ref[0, 0] = 0.0

    o_ref[0, 0] += tot

    @pl.when(i == _LOSS_G - 1)
    def _():
        o_ref[0, 0] = -o_ref[0, 0] / BATCH


@jax.jit
def _loss(crows, prows):
    return pl.pallas_call(
        _loss_body,
        grid=(_LOSS_G,),
        in_specs=[
            pl.BlockSpec((_LOSS_C, EMBED), lambda i: (i, 0)),
            pl.BlockSpec((NPART, _LOSS_C, EMBED), lambda i: (0, i, 0)),
        ],
        out_specs=pl.BlockSpec(
            (1, 1), lambda i: (0, 0), memory_space=pltpu.SMEM),
        out_shape=jax.ShapeDtypeStruct((1, 1), jnp.float32),
    )(crows, prows)


def kernel(center, context, neg_samples, in_embed_W, out_embed_W):
    center = center.astype(jnp.int32)
    pidx = jnp.concatenate(
        [context.astype(jnp.int32)[None, :],
         neg_samples.astype(jnp.int32).T], axis=0)
    cidx2d = center.reshape(BATCH // _CHUNK, _CHUNK)
    pidx3d = pidx.reshape(NPART, BATCH // _CHUNK, _CHUNK)
    crows, prows = _sc_gather(cidx2d, pidx3d, in_embed_W, out_embed_W)
    return _loss(crows, prows)[0, 0]


# speedup vs baseline: 11.1725x; 2.4347x over previous
import functools

import jax
import jax.numpy as jnp
from jax import lax
from jax.experimental import pallas as pl
from jax.experimental.pallas import tpu as pltpu
from jax.experimental.pallas import tpu_sc as plsc

VOCAB = 1000000
EMBED = 64
PAIR = 2 * EMBED
BATCH = 16384
NEG = 20
NPART = NEG + 1

_TR = 8192
_TCH = 2 * _TR
_VP_MIN = ((VOCAB + 255) // 256) * 128
_TG = (_VP_MIN + _TR - 1) // _TR
_VP = _TG * _TR

_NC = 2
_NS = 16
_NW = _NC * _NS
_BPW = BATCH // _NW
_CHUNK = 128
_BUF = 512
_GPB = _BUF // _CHUNK


def _pack_body(wt_ref, o_ref):
    x = wt_ref[...]
    eye = (lax.broadcasted_iota(jnp.int32, (EMBED, EMBED), 0) ==
           lax.broadcasted_iota(jnp.int32, (EMBED, EMBED), 1)).astype(jnp.float32)
    y = lax.dot_general(x, eye, (((0,), (0,)), ((), ())),
                        precision=lax.Precision.DEFAULT)
    y4 = y.reshape(_TCH // 256, 2, _CHUNK, EMBED)
    out = jnp.concatenate([y4[:, 0], y4[:, 1]], axis=-1)
    o_ref[...] = out.reshape(_TR, PAIR)


@jax.jit
def _pack_table(w):
    wt = jnp.transpose(w)
    return pl.pallas_call(
        _pack_body,
        grid=(_TG,),
        in_specs=[pl.BlockSpec((EMBED, _TCH), lambda i: (0, i))],
        out_specs=pl.BlockSpec((_TR, PAIR), lambda i: (i, 0)),
        out_shape=jax.ShapeDtypeStruct((_VP, PAIR), jnp.float32),
    )(wt)


def _gather_pipeline(tasks, table, idxbuf, rowbuf, isem, gsem, wsem):
    n = len(tasks)
    idx_h = [None] * n
    wr_h = [None] * n
    idx_h[0] = pltpu.async_copy(tasks[0][0], idxbuf.at[0], isem.at[0])
    for t in range(n):
        s = t % 2
        idx_h[t].wait()
        if t + 1 < n:
            idx_h[t + 1] = pltpu.async_copy(tasks[t + 1][0], idxbuf.at[1 - s],
                                            isem.at[1 - s])
        if t >= 2:
            wr_h[t - 2].wait()
        gh = []
        for j in range(_GPB):
            gh.append(pltpu.async_copy(
                table.at[idxbuf.at[s, j]],
                rowbuf.at[s, pl.ds(j * _CHUNK, _CHUNK)],
                gsem.at[s]))
        for h in gh:
            h.wait()
        wr_h[t] = pltpu.async_copy(rowbuf.at[s], tasks[t][1], wsem.at[s])
    for h in wr_h[max(0, n - 2):]:
        h.wait()


def _sc_gather_body(pidx, out_wf, prows, idxbuf, rowbuf, isem, gsem, wsem):
    wid = lax.axis_index("s") * _NC + lax.axis_index("c")
    basec = wid * (_BPW // _CHUNK)
    baser = wid * _BPW
    tasks = []
    for t in range(NPART):
        for sb in range(_BPW // _BUF):
            tasks.append((
                pidx.at[t, pl.ds(basec + sb * _GPB, _GPB)],
                prows.at[t, pl.ds(baser + sb * _BUF, _BUF)],
            ))
    _gather_pipeline(tasks, out_wf, idxbuf, rowbuf, isem, gsem, wsem)


def _sc_gather_c_body(cidx, in_wf, crows, idxbuf, rowbuf, isem, gsem, wsem):
    wid = lax.axis_index("s") * _NC + lax.axis_index("c")
    basec = wid * (_BPW // _CHUNK)
    baser = wid * _BPW
    tasks = []
    for sb in range(_BPW // _BUF):
        tasks.append((
            cidx.at[pl.ds(basec + sb * _GPB, _GPB)],
            crows.at[pl.ds(baser + sb * _BUF, _BUF)],
        ))
    _gather_pipeline(tasks, in_wf, idxbuf, rowbuf, isem, gsem, wsem)


_SC_SCRATCH = [
    pltpu.VMEM((2, _GPB, _CHUNK), jnp.int32),
    pltpu.VMEM((2, _BUF, EMBED), jnp.float32),
    pltpu.SemaphoreType.DMA((2,)),
    pltpu.SemaphoreType.DMA((2,)),
    pltpu.SemaphoreType.DMA((2,)),
]


@jax.jit
def _sc_gather(pidx, out_wf):
    mesh = plsc.VectorSubcoreMesh(core_axis_name="c", subcore_axis_name="s")
    f = functools.partial(
        pl.kernel,
        mesh=mesh,
        compiler_params=pltpu.CompilerParams(use_tc_tiling_on_sc=False),
        out_type=jax.ShapeDtypeStruct((NPART, BATCH, EMBED), jnp.float32),
        scratch_types=list(_SC_SCRATCH),
    )(_sc_gather_body)
    return f(pidx, out_wf)


@jax.jit
def _sc_gather_c(cidx, in_wf):
    mesh = plsc.VectorSubcoreMesh(core_axis_name="c", subcore_axis_name="s")
    f = functools.partial(
        pl.kernel,
        mesh=mesh,
        compiler_params=pltpu.CompilerParams(use_tc_tiling_on_sc=False),
        out_type=jax.ShapeDtypeStruct((BATCH, EMBED), jnp.float32),
        scratch_types=list(_SC_SCRATCH),
    )(_sc_gather_c_body)
    return f(cidx, in_wf)


_LOSS_C = 1024
_LOSS_G = BATCH // 2 // _LOSS_C


def _loss_body(c_ref, p_ref, o_ref):
    i = pl.program_id(0)
    c2 = c_ref[...]
    p2 = p_ref[...]
    prod = p2 * c2[None, :, :]
    hi = lax.broadcasted_iota(jnp.int32, prod.shape, 2) >= EMBED
    s_all = jnp.sum(prod, axis=-1)
    s_odd = jnp.sum(jnp.where(hi, prod, 0.0), axis=-1)
    sign = jnp.where(
        lax.broadcasted_iota(jnp.int32, s_all.shape, 0) == 0, 1.0, -1.0)
    term = (jnp.log(jax.nn.sigmoid(sign * (s_all - s_odd)) + 1e-10) +
            jnp.log(jax.nn.sigmoid(sign * s_odd) + 1e-10))
    tot = jnp.sum(term)

    @pl.when(i == 0)
    def _():
        o_ref[0, 0] = 0.0

    o_ref[0, 0] += tot

    @pl.when(i == _LOSS_G - 1)
    def _():
        o_ref[0, 0] = -o_ref[0, 0] / BATCH


@jax.jit
def _loss(crows2, prows2):
    return pl.pallas_call(
        _loss_body,
        grid=(_LOSS_G,),
        in_specs=[
            pl.BlockSpec((_LOSS_C, PAIR), lambda i: (i, 0)),
            pl.BlockSpec((NPART, _LOSS_C, PAIR), lambda i: (0, i, 0)),
        ],
        out_specs=pl.BlockSpec(
            (1, 1), lambda i: (0, 0), memory_space=pltpu.SMEM),
        out_shape=jax.ShapeDtypeStruct((1, 1), jnp.float32),
    )(crows2, prows2)


def _flat_idx(r):
    q = ((r >> 8) << 7) | (r & 127)
    return (q << 1) | ((r >> 7) & 1)


def kernel(center, context, neg_samples, in_embed_W, out_embed_W):
    center = center.astype(jnp.int32)
    pidx = jnp.concatenate(
        [context.astype(jnp.int32)[None, :],
         neg_samples.astype(jnp.int32).T], axis=0)
    cidx2d = _flat_idx(center).reshape(BATCH // _CHUNK, _CHUNK)
    pidx3d = _flat_idx(pidx).reshape(NPART, BATCH // _CHUNK, _CHUNK)
    out_w2 = _pack_table(out_embed_W)
    in_w_dep, _ = lax.optimization_barrier((in_embed_W, out_w2))
    in_w2 = _pack_table(in_w_dep)
    out_wf = jnp.reshape(out_w2, (2 * _VP, EMBED))
    in_wf = jnp.reshape(in_w2, (2 * _VP, EMBED))
    prows = _sc_gather(pidx3d, out_wf)
    crows = _sc_gather_c(cidx2d, in_wf)
    prows2 = jnp.reshape(prows, (NPART, BATCH // 2, PAIR))
    crows2 = jnp.reshape(crows, (BATCH // 2, PAIR))
    return _loss(crows2, prows2)[0, 0]


# speedup vs baseline: 11.7412x; 1.0509x over previous
import functools

import jax
import jax.numpy as jnp
from jax import lax
from jax.experimental import pallas as pl
from jax.experimental.pallas import tpu as pltpu
from jax.experimental.pallas import tpu_sc as plsc

VOCAB = 1000000
EMBED = 64
PAIR = 2 * EMBED
BATCH = 16384
NEG = 20
NPART = NEG + 1

_TR = 16384
_TCH = 2 * _TR
_VP_MIN = ((VOCAB + 255) // 256) * 128
_TG = (_VP_MIN + _TR - 1) // _TR
_VP = _TG * _TR

_NC = 2
_NS = 16
_NW = _NC * _NS
_BPW = BATCH // _NW
_CHUNK = 128
_BUF = 512
_GPB = _BUF // _CHUNK


def _pack_body(wt_ref, o_ref):
    x = wt_ref[...]
    eye = (lax.broadcasted_iota(jnp.int32, (EMBED, EMBED), 0) ==
           lax.broadcasted_iota(jnp.int32, (EMBED, EMBED), 1)).astype(jnp.float32)
    y = lax.dot_general(x, eye, (((0,), (0,)), ((), ())),
                        precision=lax.Precision.DEFAULT)
    y4 = y.reshape(_TCH // 256, 2, _CHUNK, EMBED)
    out = jnp.concatenate([y4[:, 0], y4[:, 1]], axis=-1)
    o_ref[...] = out.reshape(_TR, PAIR)


@jax.jit
def _pack_table(w):
    wt = jnp.transpose(w)
    return pl.pallas_call(
        _pack_body,
        grid=(_TG,),
        in_specs=[pl.BlockSpec((EMBED, _TCH), lambda i: (0, i))],
        out_specs=pl.BlockSpec((_TR, PAIR), lambda i: (i, 0)),
        out_shape=jax.ShapeDtypeStruct((_VP, PAIR), jnp.float32),
    )(wt)


def _gather_pipeline(tasks, table, idxbuf, rowbuf, isem, gsem, wsem):
    n = len(tasks)
    idx_h = [None] * n
    wr_h = [None] * n
    idx_h[0] = pltpu.async_copy(tasks[0][0], idxbuf.at[0], isem.at[0])
    for t in range(n):
        s = t % 2
        idx_h[t].wait()
        if t + 1 < n:
            idx_h[t + 1] = pltpu.async_copy(tasks[t + 1][0], idxbuf.at[1 - s],
                                            isem.at[1 - s])
        if t >= 2:
            wr_h[t - 2].wait()
        gh = []
        for j in range(_GPB):
            gh.append(pltpu.async_copy(
                table.at[idxbuf.at[s, j]],
                rowbuf.at[s, pl.ds(j * _CHUNK, _CHUNK)],
                gsem.at[s]))
        for h in gh:
            h.wait()
        wr_h[t] = pltpu.async_copy(rowbuf.at[s], tasks[t][1], wsem.at[s])
    for h in wr_h[max(0, n - 2):]:
        h.wait()


def _sc_gather_body(pidx, out_wf, prows, idxbuf, rowbuf, isem, gsem, wsem):
    wid = lax.axis_index("s") * _NC + lax.axis_index("c")
    basec = wid * (_BPW // _CHUNK)
    baser = wid * _BPW
    tasks = []
    for t in range(NPART):
        for sb in range(_BPW // _BUF):
            tasks.append((
                pidx.at[t, pl.ds(basec + sb * _GPB, _GPB)],
                prows.at[t, pl.ds(baser + sb * _BUF, _BUF)],
            ))
    _gather_pipeline(tasks, out_wf, idxbuf, rowbuf, isem, gsem, wsem)


def _sc_gather_c_body(cidx, in_wf, crows, idxbuf, rowbuf, isem, gsem, wsem):
    wid = lax.axis_index("s") * _NC + lax.axis_index("c")
    basec = wid * (_BPW // _CHUNK)
    baser = wid * _BPW
    tasks = []
    for sb in range(_BPW // _BUF):
        tasks.append((
            cidx.at[pl.ds(basec + sb * _GPB, _GPB)],
            crows.at[pl.ds(baser + sb * _BUF, _BUF)],
        ))
    _gather_pipeline(tasks, in_wf, idxbuf, rowbuf, isem, gsem, wsem)


_SC_SCRATCH = [
    pltpu.VMEM((2, _GPB, _CHUNK), jnp.int32),
    pltpu.VMEM((2, _BUF, EMBED), jnp.float32),
    pltpu.SemaphoreType.DMA((2,)),
    pltpu.SemaphoreType.DMA((2,)),
    pltpu.SemaphoreType.DMA((2,)),
]


@jax.jit
def _sc_gather(pidx, out_wf):
    mesh = plsc.VectorSubcoreMesh(core_axis_name="c", subcore_axis_name="s")
    f = functools.partial(
        pl.kernel,
        mesh=mesh,
        compiler_params=pltpu.CompilerParams(use_tc_tiling_on_sc=False),
        out_type=jax.ShapeDtypeStruct((NPART, BATCH, EMBED), jnp.float32),
        scratch_types=list(_SC_SCRATCH),
    )(_sc_gather_body)
    return f(pidx, out_wf)


@jax.jit
def _sc_gather_c(cidx, in_wf):
    mesh = plsc.VectorSubcoreMesh(core_axis_name="c", subcore_axis_name="s")
    f = functools.partial(
        pl.kernel,
        mesh=mesh,
        compiler_params=pltpu.CompilerParams(use_tc_tiling_on_sc=False),
        out_type=jax.ShapeDtypeStruct((BATCH, EMBED), jnp.float32),
        scratch_types=list(_SC_SCRATCH),
    )(_sc_gather_c_body)
    return f(cidx, in_wf)


_LOSS_C = 1024
_LOSS_G = BATCH // 2 // _LOSS_C


def _loss_body(c_ref, p_ref, o_ref):
    i = pl.program_id(0)
    c2 = c_ref[...]
    p2 = p_ref[...]
    prod = p2 * c2[None, :, :]
    hi = lax.broadcasted_iota(jnp.int32, prod.shape, 2) >= EMBED
    s_all = jnp.sum(prod, axis=-1)
    s_odd = jnp.sum(jnp.where(hi, prod, 0.0), axis=-1)
    sign = jnp.where(
        lax.broadcasted_iota(jnp.int32, s_all.shape, 0) == 0, 1.0, -1.0)
    term = (jnp.log(jax.nn.sigmoid(sign * (s_all - s_odd)) + 1e-10) +
            jnp.log(jax.nn.sigmoid(sign * s_odd) + 1e-10))
    tot = jnp.sum(term)

    @pl.when(i == 0)
    def _():
        o_ref[0, 0] = 0.0

    o_ref[0, 0] += tot

    @pl.when(i == _LOSS_G - 1)
    def _():
        o_ref[0, 0] = -o_ref[0, 0] / BATCH


@jax.jit
def _loss(crows2, prows2):
    return pl.pallas_call(
        _loss_body,
        grid=(_LOSS_G,),
        in_specs=[
            pl.BlockSpec((_LOSS_C, PAIR), lambda i: (i, 0)),
            pl.BlockSpec((NPART, _LOSS_C, PAIR), lambda i: (0, i, 0)),
        ],
        out_specs=pl.BlockSpec(
            (1, 1), lambda i: (0, 0), memory_space=pltpu.SMEM),
        out_shape=jax.ShapeDtypeStruct((1, 1), jnp.float32),
    )(crows2, prows2)


def _flat_idx(r):
    q = ((r >> 8) << 7) | (r & 127)
    return (q << 1) | ((r >> 7) & 1)


def kernel(center, context, neg_samples, in_embed_W, out_embed_W):
    center = center.astype(jnp.int32)
    pidx = jnp.concatenate(
        [context.astype(jnp.int32)[None, :],
         neg_samples.astype(jnp.int32).T], axis=0)
    cidx2d = _flat_idx(center).reshape(BATCH // _CHUNK, _CHUNK)
    pidx3d = _flat_idx(pidx).reshape(NPART, BATCH // _CHUNK, _CHUNK)
    out_w2 = _pack_table(out_embed_W)
    in_w_dep, _ = lax.optimization_barrier((in_embed_W, out_w2))
    in_w2 = _pack_table(in_w_dep)
    out_wf = jnp.reshape(out_w2, (2 * _VP, EMBED))
    in_wf = jnp.reshape(in_w2, (2 * _VP, EMBED))
    prows = _sc_gather(pidx3d, out_wf)
    crows = _sc_gather_c(cidx2d, in_wf)
    prows2 = jnp.reshape(prows, (NPART, BATCH // 2, PAIR))
    crows2 = jnp.reshape(crows, (BATCH // 2, PAIR))
    return _loss(crows2, prows2)[0, 0]
